# baseline (speedup 1.0000x reference)
import functools

import jax
import jax.numpy as jnp
from jax import lax
from jax.experimental import pallas as pl
from jax.experimental.pallas import tpu as pltpu
from jax.experimental.pallas import tpu_sc as plsc

N = 500000
D = 64
R = 800
CB = R * D
NCHUNK = N // R
NC = 2
NS = 16
NW = NC * NS
NB = 2
TPW = -(-NCHUNK // NW)
MPAD = 128


def _body(numbers_hbm, mapper_hbm, eye_hbm, out_hbm,
          map_v, eye_v, z_v, c0, c1, buf0, buf1, sem0, sem1):
    bufs = (buf0, buf1)
    csaves = (c0, c1)
    sems = (sem0, sem1)
    wid = lax.axis_index("s") * NC + lax.axis_index("c")
    lane64 = lax.broadcasted_iota(jnp.int32, (16,), 0) * D
    zeros16 = jnp.zeros((16,), jnp.float32)

    pltpu.sync_copy(mapper_hbm, map_v)
    pltpu.sync_copy(eye_hbm, eye_v)

    def zero_slot(i, _):
        for buf in bufs:
            buf[pl.ds(i * 16, 16)] = zeros16
        return 0

    lax.fori_loop(0, CB // 16, zero_slot, 0)

    def round_body(tt, _):
        for b in range(NB):
            chunk = wid + (NB * tt + b) * NW

            @pl.when(chunk < NCHUNK)
            def _(b=b, chunk=chunk):
                buf, c_v, sem = bufs[b], csaves[b], sems[b]
                pltpu.sync_copy(numbers_hbm.at[pl.ds(chunk * R, R)], z_v)

                @pl.when(tt > 0)
                def _():
                    pltpu.make_async_copy(buf, out_hbm.at[pl.ds(0, CB)], sem).wait()

                    def clear(j, _):
                        idx = c_v[pl.ds(j * 16, 16)]
                        plsc.store_scatter(buf, [idx], zeros16)
                        return 0

                    lax.fori_loop(0, R // 16, clear, 0)

                def fill(j, _):
                    z = z_v[pl.ds(j * 16, 16)]
                    c = plsc.load_gather(map_v, [z])
                    val = plsc.load_gather(eye_v, [c, c])
                    idx = j * (16 * D) + lane64 + c
                    plsc.store_scatter(buf, [idx], val)
                    c_v[pl.ds(j * 16, 16)] = idx
                    return 0

                lax.fori_loop(0, R // 16, fill, 0)
                pltpu.async_copy(buf, out_hbm.at[pl.ds(chunk * CB, CB)], sem)

        return 0

    lax.fori_loop(0, TPW // NB, round_body, 0)

    for b in range(NB):
        pltpu.make_async_copy(bufs[b], out_hbm.at[pl.ds(0, CB)], sems[b]).wait()


@jax.jit
def kernel(numbers, mapper, eye_matrix):
    mapper_p = jnp.zeros((MPAD,), jnp.int32).at[: mapper.shape[0]].set(mapper)
    run = functools.partial(
        pl.kernel,
        out_type=jax.ShapeDtypeStruct((N * D,), jnp.float32),
        mesh=plsc.VectorSubcoreMesh(core_axis_name="c", subcore_axis_name="s"),
        compiler_params=pltpu.CompilerParams(needs_layout_passes=False),
        scratch_types=[
            pltpu.VMEM((MPAD,), jnp.int32),
            pltpu.VMEM((D, D), jnp.float32),
            pltpu.VMEM((R,), jnp.int32),
            pltpu.VMEM((R,), jnp.int32),
            pltpu.VMEM((R,), jnp.int32),
            pltpu.VMEM((CB,), jnp.float32),
            pltpu.VMEM((CB,), jnp.float32),
            pltpu.SemaphoreType.DMA,
            pltpu.SemaphoreType.DMA,
        ],
    )(_body)
    return run(numbers, mapper_p, eye_matrix).reshape(N, D)


# speedup vs baseline: 1.3586x; 1.3586x over previous
import functools

import jax
import jax.numpy as jnp
from jax import lax
from jax.experimental import pallas as pl
from jax.experimental.pallas import tpu as pltpu
from jax.experimental.pallas import tpu_sc as plsc

N = 500000
D = 64
R = 400
NCHUNK = N // R
NC = 2
NS = 16
NW = NC * NS
NB = 2
TPW = -(-NCHUNK // NW)
MPAD = 128


def _body(numbers_hbm, mapper_hbm, eye_hbm, out_hbm,
          map_v, eye_v, z_v, c0, c1, buf0, buf1, sem0, sem1):
    bufs = (buf0, buf1)
    csaves = (c0, c1)
    sems = (sem0, sem1)
    wid = lax.axis_index("s") * NC + lax.axis_index("c")
    lane = lax.broadcasted_iota(jnp.int32, (16,), 0)
    zeros16 = jnp.zeros((16,), jnp.float32)

    pltpu.sync_copy(mapper_hbm, map_v)
    pltpu.sync_copy(eye_hbm, eye_v)

    def zero_row(i, _):
        for buf in bufs:
            for k in range(D // 16):
                buf[i, pl.ds(k * 16, 16)] = zeros16
        return 0

    lax.fori_loop(0, R, zero_row, 0)

    def round_body(tt, _):
        for b in range(NB):
            chunk = wid + (NB * tt + b) * NW

            @pl.when(chunk < NCHUNK)
            def _(b=b, chunk=chunk):
                buf, c_v, sem = bufs[b], csaves[b], sems[b]
                base = chunk * R
                pltpu.sync_copy(numbers_hbm.at[pl.ds(base, R)], z_v)

                @pl.when(tt > 0)
                def _():
                    pltpu.make_async_copy(buf, out_hbm.at[pl.ds(0, R)], sem).wait()

                    def clear(j, _):
                        c = c_v[pl.ds(j * 16, 16)]
                        row = j * 16 + lane
                        plsc.store_scatter(buf, [row, c], zeros16)
                        return 0

                    lax.fori_loop(0, R // 16, clear, 0)

                def fill(j, _):
                    z = z_v[pl.ds(j * 16, 16)]
                    c = plsc.load_gather(map_v, [z])
                    row = j * 16 + lane
                    val = plsc.load_gather(eye_v, [c, c])
                    plsc.store_scatter(buf, [row, c], val)
                    c_v[pl.ds(j * 16, 16)] = c
                    return 0

                lax.fori_loop(0, R // 16, fill, 0)
                pltpu.async_copy(buf, out_hbm.at[pl.ds(base, R)], sem)

        return 0

    lax.fori_loop(0, TPW // NB, round_body, 0)

    for b in range(NB):
        pltpu.make_async_copy(bufs[b], out_hbm.at[pl.ds(0, R)], sems[b]).wait()


@jax.jit
def kernel(numbers, mapper, eye_matrix):
    mapper_p = jnp.zeros((MPAD,), jnp.int32).at[: mapper.shape[0]].set(mapper)
    run = functools.partial(
        pl.kernel,
        out_type=jax.ShapeDtypeStruct((N, D), jnp.float32),
        mesh=plsc.VectorSubcoreMesh(core_axis_name="c", subcore_axis_name="s"),
        compiler_params=pltpu.CompilerParams(needs_layout_passes=False),
        scratch_types=[
            pltpu.VMEM((MPAD,), jnp.int32),
            pltpu.VMEM((D, D), jnp.float32),
            pltpu.VMEM((R,), jnp.int32),
            pltpu.VMEM((R,), jnp.int32),
            pltpu.VMEM((R,), jnp.int32),
            pltpu.VMEM((R, D), jnp.float32),
            pltpu.VMEM((R, D), jnp.float32),
            pltpu.SemaphoreType.DMA,
            pltpu.SemaphoreType.DMA,
        ],
    )(_body)
    return run(numbers, mapper_p, eye_matrix)
